# baseline (speedup 1.0000x reference)
import functools

import jax
import jax.numpy as jnp
from jax import lax
from jax.experimental import pallas as pl
from jax.experimental.pallas import tpu as pltpu
from jax.experimental.pallas import tpu_sc as plsc

B = 16384
D = 64
NC = 2
NS = 16
NW = NC * NS
BPW = B // NW
KCH = BPW // 128

_mesh = plsc.VectorSubcoreMesh(core_axis_name="c", subcore_axis_name="s")


@functools.partial(
    pl.kernel,
    mesh=_mesh,
    compiler_params=pltpu.CompilerParams(use_tc_tiling_on_sc=False),
    out_type=[
        jax.ShapeDtypeStruct((B, D), jnp.float32),
        jax.ShapeDtypeStruct((B, D), jnp.float32),
    ],
    scratch_types=[
        pltpu.VMEM((KCH, 128), jnp.int32),
        pltpu.VMEM((KCH, 128), jnp.int32),
        pltpu.VMEM((BPW, D), jnp.float32),
        pltpu.VMEM((BPW, D), jnp.float32),
        pltpu.SemaphoreType.DMA,
    ],
)
def _sc_gather(uid_hbm, iid_hbm, ut_hbm, it_hbm, uout_hbm, iout_hbm,
               uidx_v, iidx_v, urows_v, irows_v, sem):
    wid = lax.axis_index("s") * NC + lax.axis_index("c")
    pltpu.sync_copy(uid_hbm.at[wid], uidx_v)
    pltpu.sync_copy(iid_hbm.at[wid], iidx_v)
    copies = []
    for k in range(KCH):
        copies.append(pltpu.async_copy(
            ut_hbm.at[uidx_v.at[k]], urows_v.at[pl.ds(k * 128, 128)], sem))
        copies.append(pltpu.async_copy(
            it_hbm.at[iidx_v.at[k]], irows_v.at[pl.ds(k * 128, 128)], sem))
    for c in copies:
        c.wait()
    base = wid * BPW
    pltpu.sync_copy(urows_v, uout_hbm.at[pl.ds(base, BPW)])
    pltpu.sync_copy(irows_v, iout_hbm.at[pl.ds(base, BPW)])


def _head_body(u_ref, v_ref, w_ref, b_ref, o_ref):
    m = u_ref[...] * v_ref[...]
    p = jnp.sum(m * w_ref[...], axis=1, keepdims=True) + b_ref[0]
    o_ref[...] = 1.0 / (1.0 + jnp.exp(-p))


def _head(u, v, w_row, b):
    blk = 2048
    return pl.pallas_call(
        _head_body,
        grid=(B // blk,),
        in_specs=[
            pl.BlockSpec((blk, D), lambda i: (i, 0)),
            pl.BlockSpec((blk, D), lambda i: (i, 0)),
            pl.BlockSpec((1, D), lambda i: (0, 0)),
            pl.BlockSpec(memory_space=pltpu.SMEM),
        ],
        out_specs=pl.BlockSpec((blk, 1), lambda i: (i, 0)),
        out_shape=jax.ShapeDtypeStruct((B, 1), jnp.float32),
    )(u, v, w_row, b)


def kernel(user_id, item_id, user_table, item_table, W, b):
    uid = user_id.reshape(NW, KCH, 128).astype(jnp.int32)
    iid = item_id.reshape(NW, KCH, 128).astype(jnp.int32)
    u_rows, i_rows = _sc_gather(uid, iid, user_table, item_table)
    return _head(u_rows, i_rows, W.reshape(1, D), b)
